# Initial kernel scaffold; baseline (speedup 1.0000x reference)
import jax
import jax.numpy as jnp
from jax.experimental import pallas as pl


def kernel(xs, data):
    raise NotImplementedError("write your pallas kernel here")


# speedup vs baseline: 27.4059x; 27.4059x over previous
import functools

import jax
import jax.numpy as jnp
from jax import lax
from jax.experimental import pallas as pl
from jax.experimental.pallas import tpu as pltpu
from jax.experimental.pallas import tpu_sc as plsc

H, W, C = 32, 32, 3
LANES = 16


def _sc_kernel(n_points, num_workers, chunk):
  pw = n_points // num_workers
  n_chunks = pw // chunk
  groups = chunk // LANES

  mesh = plsc.VectorSubcoreMesh(
      core_axis_name="c", subcore_axis_name="s", num_cores=2, num_subcores=16)

  @functools.partial(
      pl.kernel,
      out_type=jax.ShapeDtypeStruct((n_points * C,), jnp.float32),
      mesh=mesh,
      compiler_params=pltpu.CompilerParams(needs_layout_passes=False),
      scratch_types=[
          pltpu.VMEM((H * W * C,), jnp.float32),
          pltpu.VMEM((2 * chunk,), jnp.float32),
          pltpu.VMEM((C * chunk,), jnp.float32),
      ],
  )
  def body(xs_hbm, tab_hbm, out_hbm, tab_v, xs_v, out_v):
    wid = lax.axis_index("s") * 2 + lax.axis_index("c")
    base_pt = wid * pw
    pltpu.sync_copy(tab_hbm, tab_v)

    iota = lax.iota(jnp.int32, LANES)
    idx2 = iota * 2
    idx3 = iota * 3

    def do_chunk(ci, _):
      pt0 = base_pt + ci * chunk
      pltpu.sync_copy(xs_hbm.at[pl.ds(pt0 * 2, 2 * chunk)], xs_v)

      def do_group(g, _):
        off = g * (2 * LANES)
        ix = idx2 + off
        x = plsc.load_gather(xs_v, [ix])
        y = plsc.load_gather(xs_v, [ix + 1])
        xp = x * float(W)
        yp = y * float(H)
        xi = xp.astype(jnp.int32)
        yi = yp.astype(jnp.int32)
        wx = xp - xi.astype(jnp.float32)
        wy = yp - yi.astype(jnp.float32)
        x0 = jnp.minimum(jnp.maximum(xi, 0), W - 1)
        y0 = jnp.minimum(jnp.maximum(yi, 0), H - 1)
        x1 = jnp.minimum(x0 + 1, W - 1)
        y1 = jnp.minimum(y0 + 1, H - 1)
        cx0 = x0 * C
        cx1 = x1 * C
        ry0 = y0 * (W * C)
        ry1 = y1 * (W * C)
        p00 = ry0 + cx0
        p10 = ry0 + cx1
        p01 = ry1 + cx0
        p11 = ry1 + cx1
        one = jnp.float32(1.0)
        w00 = (one - wx) * (one - wy)
        w10 = wx * (one - wy)
        w01 = (one - wx) * wy
        w11 = wx * wy
        obase = idx3 + g * (C * LANES)
        for c in range(C):
          v00 = plsc.load_gather(tab_v, [p00 + c])
          v10 = plsc.load_gather(tab_v, [p10 + c])
          v01 = plsc.load_gather(tab_v, [p01 + c])
          v11 = plsc.load_gather(tab_v, [p11 + c])
          r = w00 * v00 + w10 * v10 + w01 * v01 + w11 * v11
          plsc.store_scatter(out_v, [obase + c], r)
        return 0

      lax.fori_loop(0, groups, do_group, 0)
      pltpu.sync_copy(out_v, out_hbm.at[pl.ds(pt0 * C, C * chunk)])
      return 0

    lax.fori_loop(0, n_chunks, do_chunk, 0)

  return body


def kernel(xs, data):
  n = xs.shape[0]
  num_workers = 32
  chunk = 2048
  assert n % (num_workers * chunk) == 0
  xs_flat = xs.reshape(-1)
  tab = data.reshape(-1)
  out_flat = _sc_kernel(n, num_workers, chunk)(xs_flat, tab)
  return out_flat.reshape(n, C)


# speedup vs baseline: 2625.0000x; 95.7823x over previous
import functools

import jax
import jax.numpy as jnp
from jax import lax
from jax.experimental import pallas as pl
from jax.experimental.pallas import tpu as pltpu
from jax.experimental.pallas import tpu_sc as plsc

H, W, C = 32, 32, 3
HP, WP = H + 1, W + 1
LANES = 16
BLK = 128
GPB = BLK // LANES


def _sc_kernel(n_points, num_workers, cblk):
  nb = n_points // BLK
  bw = nb // num_workers
  n_chunks = bw // cblk

  mesh = plsc.VectorSubcoreMesh(
      core_axis_name="c", subcore_axis_name="s", num_cores=2, num_subcores=16)

  @functools.partial(
      pl.kernel,
      out_type=jax.ShapeDtypeStruct((nb * 4 * BLK,), jnp.float32),
      mesh=mesh,
      compiler_params=pltpu.CompilerParams(needs_layout_passes=False),
      scratch_types=[
          pltpu.VMEM((HP * WP,), jnp.float32),
          pltpu.VMEM((HP * WP,), jnp.float32),
          pltpu.VMEM((HP * WP,), jnp.float32),
          pltpu.VMEM((2, cblk * 2 * BLK), jnp.float32),
          pltpu.VMEM((2, cblk * 4 * BLK), jnp.float32),
          pltpu.SemaphoreType.DMA((2,)),
          pltpu.SemaphoreType.DMA((2,)),
      ],
  )
  def body(xs_hbm, tr_hbm, tg_hbm, tb_hbm, out_hbm,
           tr_v, tg_v, tb_v, xs_v, out_v, in_sems, out_sems):
    wid = lax.axis_index("s") * 2 + lax.axis_index("c")
    base_blk = wid * bw
    pltpu.sync_copy(tr_hbm, tr_v)
    pltpu.sync_copy(tg_hbm, tg_v)
    pltpu.sync_copy(tb_hbm, tb_v)

    def in_copy(ci, b):
      return pltpu.make_async_copy(
          xs_hbm.at[pl.ds((base_blk + ci * cblk) * (2 * BLK), cblk * 2 * BLK)],
          xs_v.at[b], in_sems.at[b])

    def out_copy(ci, b):
      return pltpu.make_async_copy(
          out_v.at[b],
          out_hbm.at[pl.ds((base_blk + ci * cblk) * (4 * BLK), cblk * 4 * BLK)],
          out_sems.at[b])

    in_copy(0, 0).start()
    in_copy(1, 1).start()

    def do_chunk(ci, b):
      in_copy(ci, b).wait()

      @pl.when(ci >= 2)
      def _():
        out_copy(ci - 2, b).wait()

      xbuf = xs_v.at[b]
      obuf = out_v.at[b]

      @functools.partial(plsc.parallel_loop, 0, cblk, unroll=2)
      def _blk(blk):
        xo = blk * (2 * BLK)
        oo = blk * (4 * BLK)
        for l in range(GPB):
          x = xbuf[pl.ds(xo + l * LANES, LANES)]
          y = xbuf[pl.ds(xo + BLK + l * LANES, LANES)]
          xp = x * float(W)
          yp = y * float(H)
          xi = xp.astype(jnp.int32)
          yi = yp.astype(jnp.int32)
          wx = xp - xi.astype(jnp.float32)
          wy = yp - yi.astype(jnp.float32)
          q00 = yi * WP + xi
          q10 = q00 + 1
          q01 = q00 + WP
          q11 = q00 + (WP + 1)
          gx = jnp.float32(1.0) - wx
          gy = jnp.float32(1.0) - wy
          w00 = gx * gy
          w10 = wx * gy
          w01 = gx * wy
          w11 = wx * wy
          for c, t in ((0, tr_v), (1, tg_v), (2, tb_v)):
            v00 = plsc.load_gather(t, [q00])
            v10 = plsc.load_gather(t, [q10])
            v01 = plsc.load_gather(t, [q01])
            v11 = plsc.load_gather(t, [q11])
            r = w00 * v00 + w10 * v10 + w01 * v01 + w11 * v11
            obuf[pl.ds(oo + c * BLK + l * LANES, LANES)] = r

      out_copy(ci, b).start()

      @pl.when(ci + 2 < n_chunks)
      def _():
        in_copy(ci + 2, b).start()

      return 0

    def outer(cio, carry):
      do_chunk(cio * 2, 0)
      do_chunk(cio * 2 + 1, 1)
      return carry

    lax.fori_loop(0, n_chunks // 2, outer, 0)
    out_copy(n_chunks - 2, 0).wait()
    out_copy(n_chunks - 1, 1).wait()

  return body


def kernel(xs, data):
  n = xs.shape[0]
  num_workers = 32
  cblk = 64
  nb = n // BLK
  assert n % (num_workers * 2 * cblk * BLK) == 0
  xst = jnp.transpose(xs.reshape(nb, BLK, 2), (0, 2, 1)).reshape(-1)
  padded = jnp.pad(data, ((0, 1), (0, 1), (0, 0)), mode="edge")
  tabs = [padded[:, :, c].reshape(-1) for c in range(C)]
  out_flat = _sc_kernel(n, num_workers, cblk)(xst, *tabs)
  out3 = out_flat.reshape(nb, 4, BLK)
  return jnp.transpose(out3, (0, 2, 1))[:, :, :C].reshape(n, C)
